# Initial kernel scaffold; baseline (speedup 1.0000x reference)
import functools

import jax
import jax.numpy as jnp
from jax import lax
from jax.experimental import pallas as pl
from jax.experimental.pallas import tpu as pltpu
from jax.experimental.pallas import tpu_sc as plsc

N = 10000
E = 320000
F = 128
L = 3

LANES = 16
NC = 2
NS = 16
NW = NC * NS

C = 128
KCH = -(-E // (NW * C))
EPW = KCH * C
EPAD = NW * EPW
NPAD = N + LANES
RPT = NPAD // NS

_f32 = jnp.float32


def _agg_body(h_hbm, src_hbm, dst_hbm, attr_hbm, ecw_hbm, ecb_hbm, out_hbm,
              sidx_v, didx_v, attr_v, rows_v, w_v, ecb_v, agg_sh, gsem):
    c = lax.axis_index("c")
    s = lax.axis_index("s")
    wid = c * NS + s
    base = wid * EPW
    r0 = s * RPT

    pltpu.sync_copy(ecw_hbm, w_v)
    pltpu.sync_copy(ecb_hbm, ecb_v)
    wv = [[w_v[k, pl.ds(16 * j, 16)] for j in range(8)] for k in range(4)]
    ebv = [ecb_v[pl.ds(16 * j, 16)] for j in range(8)]

    zeros16 = jnp.zeros((LANES,), _f32)

    @pl.loop(0, C)
    def _zr(r):
        for j in range(8):
            rows_v[r, pl.ds(16 * j, 16)] = zeros16

    for t in range(RPT // C):
        pltpu.sync_copy(rows_v, agg_sh.at[pl.ds(r0 + t * C, C)])
    rem = RPT % C
    if rem:
        pltpu.sync_copy(rows_v.at[pl.ds(0, rem)],
                        agg_sh.at[pl.ds(r0 + (RPT // C) * C, rem)])
    plsc.subcore_barrier()

    @pl.loop(0, KCH)
    def _chunk(k):
        off = pl.multiple_of(base + k * C, C)
        pltpu.sync_copy(src_hbm.at[pl.ds(off, C)], sidx_v)
        pltpu.sync_copy(dst_hbm.at[pl.ds(off, C)], didx_v)
        pltpu.sync_copy(attr_hbm.at[pl.ds(off, C)], attr_v)
        pltpu.async_copy(h_hbm.at[sidx_v], rows_v, gsem).wait()

        @pl.loop(0, C)
        def _edge(e):
            e_s = jnp.full((LANES,), e, jnp.int32)
            a = [plsc.load_gather(
                    attr_v, [e_s, jnp.full((LANES,), k4, jnp.int32)])
                 for k4 in range(4)]
            for j in range(8):
                sl = pl.ds(16 * j, 16)
                val = rows_v[e, sl] + ebv[j]
                for k4 in range(4):
                    val = val + a[k4] * wv[k4][j]
                rows_v[e, sl] = jnp.maximum(val, 0.0)

        pltpu.sync_copy(rows_v, agg_sh.at[didx_v], add=True)

    plsc.subcore_barrier()
    pltpu.sync_copy(agg_sh.at[pl.ds(r0, RPT)], out_hbm.at[c, pl.ds(r0, RPT)])


_agg = functools.partial(
    pl.kernel,
    out_type=jax.ShapeDtypeStruct((NC, NPAD, F), _f32),
    mesh=plsc.VectorSubcoreMesh(core_axis_name="c", subcore_axis_name="s"),
    scratch_types=[
        pltpu.VMEM((C,), jnp.int32),
        pltpu.VMEM((C,), jnp.int32),
        pltpu.VMEM((C, 4), _f32),
        pltpu.VMEM((C, F), _f32),
        pltpu.VMEM((4, F), _f32),
        pltpu.VMEM((F,), _f32),
        pltpu.VMEM_SHARED((NPAD, F), _f32),
        pltpu.SemaphoreType.DMA,
    ],
)(_agg_body)


def _fc_body(x_ref, w_ref, o_ref):
    o_ref[...] = jnp.dot(x_ref[...], w_ref[...], preferred_element_type=_f32)


def _mlp_body(h_ref, p0_ref, p1_ref, eps_ref, w1_ref, b1_ref, g1_ref, bb1_ref,
              w2_ref, b2_ref, g2_ref, bb2_ref, o_ref, *, last):
    z = eps_ref[...] * h_ref[...] + (p0_ref[...] + p1_ref[...])
    z = jnp.dot(z, w1_ref[...], preferred_element_type=_f32) + b1_ref[...]
    m = jnp.mean(z, axis=0, keepdims=True)
    v = jnp.mean((z - m) ** 2, axis=0, keepdims=True)
    z = (z - m) * lax.rsqrt(v + 1e-5) * g1_ref[...] + bb1_ref[...]
    z = jnp.maximum(z, 0.0)
    z = jnp.dot(z, w2_ref[...], preferred_element_type=_f32) + b2_ref[...]
    m2 = jnp.mean(z, axis=0, keepdims=True)
    v2 = jnp.mean((z - m2) ** 2, axis=0, keepdims=True)
    z = (z - m2) * lax.rsqrt(v2 + 1e-5) * g2_ref[...] + bb2_ref[...]
    if not last:
        z = jnp.maximum(z, 0.0)
    o_ref[...] = z


def kernel(x, edge_index, edge_attr, batch, fc_w, ec_w, ec_b, mlp1_w, mlp1_b,
           bn1_g, bn1_b, mlp2_w, mlp2_b, gin_eps, bn_g, bn_b):
    src = edge_index[0]
    dst = edge_index[1]
    pad = EPAD - E
    src_p = jnp.concatenate([src, jnp.zeros((pad,), jnp.int32)])
    dst_p = jnp.concatenate([dst, jnp.full((pad,), N, jnp.int32)])
    attr_p = jnp.concatenate([edge_attr, jnp.zeros((pad, 4), _f32)])

    h = pl.pallas_call(
        _fc_body, out_shape=jax.ShapeDtypeStruct((N, F), _f32))(x, fc_w)

    for l in range(L):
        parts = _agg(h, src_p, dst_p, attr_p, ec_w[l], ec_b[l])
        mlp = pl.pallas_call(
            functools.partial(_mlp_body, last=(l == L - 1)),
            out_shape=jax.ShapeDtypeStruct((N, F), _f32))
        h = mlp(h, parts[0, :N], parts[1, :N],
                (1.0 + gin_eps[l]).reshape(1, 1),
                mlp1_w[l], mlp1_b[l].reshape(1, -1),
                bn1_g[l].reshape(1, -1), bn1_b[l].reshape(1, -1),
                mlp2_w[l], mlp2_b[l].reshape(1, -1),
                bn_g[l].reshape(1, -1), bn_b[l].reshape(1, -1))
    return h


# speedup vs baseline: 1.2092x; 1.2092x over previous
import functools

import jax
import jax.numpy as jnp
from jax import lax
from jax.experimental import pallas as pl
from jax.experimental.pallas import tpu as pltpu
from jax.experimental.pallas import tpu_sc as plsc

N = 10000
E = 320000
F = 128
L = 3

LANES = 16
NC = 2
NS = 16
NW = NC * NS

C = 128
KCH = -(-E // (NW * C))
EPW = KCH * C
EPAD = NW * EPW
NPAD = N + NS * 8 - N % (NS * 8)
RPT = NPAD // NS

_f32 = jnp.float32


def _agg_body(h_hbm, src_hbm, dst_hbm, attr_hbm, ecw_hbm, ecb_hbm, out_hbm,
              sidx_v, didx_v, attr_v, rows_v, w_v, ecb_v, agg_sh, gsem):
    c = lax.axis_index("c")
    s = lax.axis_index("s")
    wid = c * NS + s
    base = wid * EPW
    r0 = s * RPT

    pltpu.sync_copy(ecw_hbm, w_v)
    pltpu.sync_copy(ecb_hbm, ecb_v)
    wv = [[w_v[k, pl.ds(16 * j, 16)] for j in range(8)] for k in range(4)]
    ebv = [ecb_v[pl.ds(16 * j, 16)] for j in range(8)]

    zeros16 = jnp.zeros((LANES,), _f32)

    @pl.loop(0, C)
    def _zr(r):
        for j in range(8):
            rows_v[r, pl.ds(16 * j, 16)] = zeros16

    for t in range(RPT // C):
        pltpu.sync_copy(rows_v, agg_sh.at[pl.ds(r0 + t * C, C)])
    rem = RPT % C
    if rem:
        pltpu.sync_copy(rows_v.at[pl.ds(0, rem)],
                        agg_sh.at[pl.ds(r0 + (RPT // C) * C, rem)])
    plsc.subcore_barrier()

    @pl.loop(0, KCH)
    def _chunk(k):
        off = pl.multiple_of(base + k * C, C)
        pltpu.sync_copy(src_hbm.at[pl.ds(off, C)], sidx_v)
        pltpu.sync_copy(dst_hbm.at[pl.ds(off, C)], didx_v)
        pltpu.sync_copy(attr_hbm.at[pl.ds(off * 4, C * 4)], attr_v)
        pltpu.async_copy(h_hbm.at[sidx_v], rows_v, gsem).wait()

        dn = lax.GatherDimensionNumbers(
            offset_dims=(), collapsed_slice_dims=(0,), start_index_map=(0,))

        @pl.loop(0, C // 4)
        def _grp(g):
            av = attr_v[pl.ds(g * LANES, LANES)]
            for t in range(4):
                e = g * 4 + t
                a = [lax.gather(av, jnp.full((LANES, 1), 4 * t + k4,
                                             jnp.int32),
                                dn, (1,),
                                mode=lax.GatherScatterMode.PROMISE_IN_BOUNDS)
                     for k4 in range(4)]
                for j in range(8):
                    sl = pl.ds(16 * j, 16)
                    val = rows_v[e, sl] + ebv[j]
                    for k4 in range(4):
                        val = val + a[k4] * wv[k4][j]
                    rows_v[e, sl] = jnp.maximum(val, 0.0)

        pltpu.sync_copy(rows_v, agg_sh.at[didx_v], add=True)

    plsc.subcore_barrier()
    pltpu.sync_copy(agg_sh.at[pl.ds(r0, RPT)], out_hbm.at[c, pl.ds(r0, RPT)])


@functools.cache
def _agg():
    return functools.partial(
        pl.kernel,
        out_type=jax.ShapeDtypeStruct((NC, NPAD, F), _f32),
        mesh=plsc.VectorSubcoreMesh(core_axis_name="c", subcore_axis_name="s",
                                    num_cores=NC, num_subcores=NS),
        scratch_types=[
            pltpu.VMEM((C,), jnp.int32),
            pltpu.VMEM((C,), jnp.int32),
            pltpu.VMEM((C * 4,), _f32),
            pltpu.VMEM((C, F), _f32),
            pltpu.VMEM((4, F), _f32),
            pltpu.VMEM((F,), _f32),
            pltpu.VMEM_SHARED((NPAD, F), _f32),
            pltpu.SemaphoreType.DMA,
        ],
    )(_agg_body)


def _dot(a, b):
    return jnp.dot(a, b, preferred_element_type=_f32)


def _fc_body(x_ref, w_ref, o_ref):
    o_ref[...] = _dot(x_ref[...], w_ref[...])


def _mlp_body(h_ref, p0_ref, p1_ref, eps_ref, w1_ref, b1_ref, g1_ref, bb1_ref,
              w2_ref, b2_ref, g2_ref, bb2_ref, o_ref, *, last):
    z = eps_ref[...] * h_ref[...] + (p0_ref[...] + p1_ref[...])
    z = _dot(z, w1_ref[...]) + b1_ref[...]
    m = jnp.mean(z, axis=0, keepdims=True)
    v = jnp.mean((z - m) ** 2, axis=0, keepdims=True)
    z = (z - m) / jnp.sqrt(v + 1e-5) * g1_ref[...] + bb1_ref[...]
    z = jnp.maximum(z, 0.0)
    z = _dot(z, w2_ref[...]) + b2_ref[...]
    m2 = jnp.mean(z, axis=0, keepdims=True)
    v2 = jnp.mean((z - m2) ** 2, axis=0, keepdims=True)
    z = (z - m2) / jnp.sqrt(v2 + 1e-5) * g2_ref[...] + bb2_ref[...]
    if not last:
        z = jnp.maximum(z, 0.0)
    o_ref[...] = z


def kernel(x, edge_index, edge_attr, batch, fc_w, ec_w, ec_b, mlp1_w, mlp1_b,
           bn1_g, bn1_b, mlp2_w, mlp2_b, gin_eps, bn_g, bn_b):
    src = edge_index[0]
    dst = edge_index[1]
    order = jnp.argsort(dst, stable=True)
    src = src[order]
    dst = dst[order]
    edge_attr = edge_attr[order]
    pad = EPAD - E
    src_p = jnp.concatenate([src, jnp.zeros((pad,), jnp.int32)])
    dst_p = jnp.concatenate([dst, jnp.full((pad,), N, jnp.int32)])
    attr_p = jnp.concatenate([edge_attr, jnp.zeros((pad, 4), _f32)]).reshape(-1)

    h = pl.pallas_call(
        _fc_body, out_shape=jax.ShapeDtypeStruct((N, F), _f32))(x, fc_w)

    for l in range(L):
        parts = _agg()(h, src_p, dst_p, attr_p, ec_w[l], ec_b[l])
        mlp = pl.pallas_call(
            functools.partial(_mlp_body, last=(l == L - 1)),
            out_shape=jax.ShapeDtypeStruct((N, F), _f32))
        h = mlp(h, parts[0, :N], parts[1, :N],
                (1.0 + gin_eps[l]).reshape(1, 1),
                mlp1_w[l], mlp1_b[l].reshape(1, -1),
                bn1_g[l].reshape(1, -1), bn1_b[l].reshape(1, -1),
                mlp2_w[l], mlp2_b[l].reshape(1, -1),
                bn_g[l].reshape(1, -1), bn_b[l].reshape(1, -1))
    return h


# speedup vs baseline: 1.8042x; 1.4921x over previous
import functools

import jax
import jax.numpy as jnp
from jax import lax
from jax.experimental import pallas as pl
from jax.experimental.pallas import tpu as pltpu
from jax.experimental.pallas import tpu_sc as plsc

N = 10000
E = 320000
F = 128
L = 3

LANES = 16
NC = 2
NS = 16
NW = NC * NS

C = 128
KCH = -(-E // (NW * C))
EPW = KCH * C
EPAD = NW * EPW
NPAD = N + NS * 8 - N % (NS * 8)
RPT = NPAD // NS

_f32 = jnp.float32


def _agg_body(h_hbm, src_hbm, dst_hbm, ee_hbm, out_hbm,
              sidx_v, didx_v, ee_v, rows_v, agg_sh, gsem):
    c = lax.axis_index("c")
    s = lax.axis_index("s")
    wid = c * NS + s
    base = wid * EPW
    r0 = s * RPT

    zeros16 = jnp.zeros((LANES,), _f32)

    @pl.loop(0, C)
    def _zr(r):
        for j in range(8):
            rows_v[r, pl.ds(16 * j, 16)] = zeros16

    for t in range(RPT // C):
        pltpu.sync_copy(rows_v, agg_sh.at[pl.ds(r0 + t * C, C)])
    rem = RPT % C
    if rem:
        pltpu.sync_copy(rows_v.at[pl.ds(0, rem)],
                        agg_sh.at[pl.ds(r0 + (RPT // C) * C, rem)])
    plsc.subcore_barrier()

    @pl.loop(0, KCH)
    def _chunk(k):
        off = pl.multiple_of(base + k * C, C)
        pltpu.sync_copy(src_hbm.at[pl.ds(off, C)], sidx_v)
        pltpu.sync_copy(dst_hbm.at[pl.ds(off, C)], didx_v)
        pltpu.sync_copy(ee_hbm.at[pl.ds(off, C)], ee_v)
        pltpu.async_copy(h_hbm.at[sidx_v], rows_v, gsem).wait()

        @pl.loop(0, C)
        def _edge(e):
            for j in range(8):
                sl = pl.ds(16 * j, 16)
                rows_v[e, sl] = jnp.maximum(rows_v[e, sl] + ee_v[e, sl], 0.0)

        pltpu.sync_copy(rows_v, agg_sh.at[didx_v], add=True)

    plsc.subcore_barrier()
    pltpu.sync_copy(agg_sh.at[pl.ds(r0, RPT)], out_hbm.at[c, pl.ds(r0, RPT)])


@functools.cache
def _agg():
    return functools.partial(
        pl.kernel,
        out_type=jax.ShapeDtypeStruct((NC, NPAD, F), _f32),
        mesh=plsc.VectorSubcoreMesh(core_axis_name="c", subcore_axis_name="s",
                                    num_cores=NC, num_subcores=NS),
        scratch_types=[
            pltpu.VMEM((C,), jnp.int32),
            pltpu.VMEM((C,), jnp.int32),
            pltpu.VMEM((C, F), _f32),
            pltpu.VMEM((C, F), _f32),
            pltpu.VMEM_SHARED((NPAD, F), _f32),
            pltpu.SemaphoreType.DMA,
        ],
    )(_agg_body)


def _dot(a, b):
    return jnp.dot(a, b, preferred_element_type=_f32)


def _fc_body(x_ref, w_ref, o_ref):
    o_ref[...] = _dot(x_ref[...], w_ref[...])


def _ee_body(a_ref, w_ref, b_ref, o_ref):
    o_ref[...] = _dot(a_ref[...], w_ref[...]) + b_ref[...]


def _mlp_body(h_ref, p0_ref, p1_ref, eps_ref, w1_ref, b1_ref, g1_ref, bb1_ref,
              w2_ref, b2_ref, g2_ref, bb2_ref, o_ref, *, last):
    z = eps_ref[...] * h_ref[...] + (p0_ref[...] + p1_ref[...])
    z = _dot(z, w1_ref[...]) + b1_ref[...]
    m = jnp.mean(z, axis=0, keepdims=True)
    v = jnp.mean((z - m) ** 2, axis=0, keepdims=True)
    z = (z - m) / jnp.sqrt(v + 1e-5) * g1_ref[...] + bb1_ref[...]
    z = jnp.maximum(z, 0.0)
    z = _dot(z, w2_ref[...]) + b2_ref[...]
    m2 = jnp.mean(z, axis=0, keepdims=True)
    v2 = jnp.mean((z - m2) ** 2, axis=0, keepdims=True)
    z = (z - m2) / jnp.sqrt(v2 + 1e-5) * g2_ref[...] + bb2_ref[...]
    if not last:
        z = jnp.maximum(z, 0.0)
    o_ref[...] = z


def kernel(x, edge_index, edge_attr, batch, fc_w, ec_w, ec_b, mlp1_w, mlp1_b,
           bn1_g, bn1_b, mlp2_w, mlp2_b, gin_eps, bn_g, bn_b):
    src = edge_index[0]
    dst = edge_index[1]
    order = jnp.argsort(dst, stable=True)
    src = src[order]
    dst = dst[order]
    edge_attr = edge_attr[order]
    pad = EPAD - E
    src_p = jnp.concatenate([src, jnp.zeros((pad,), jnp.int32)])
    dst_p = jnp.concatenate([dst, jnp.full((pad,), N, jnp.int32)])
    attr_p = jnp.concatenate([edge_attr, jnp.zeros((pad, 4), _f32)])

    h = pl.pallas_call(
        _fc_body, out_shape=jax.ShapeDtypeStruct((N, F), _f32))(x, fc_w)

    for l in range(L):
        ee = pl.pallas_call(
            _ee_body,
            grid=(EPAD // 4096,),
            in_specs=[pl.BlockSpec((4096, 4), lambda i: (i, 0)),
                      pl.BlockSpec((4, F), lambda i: (0, 0)),
                      pl.BlockSpec((1, F), lambda i: (0, 0))],
            out_specs=pl.BlockSpec((4096, F), lambda i: (i, 0)),
            out_shape=jax.ShapeDtypeStruct((EPAD, F), _f32))(
                attr_p, ec_w[l], ec_b[l].reshape(1, -1))
        parts = _agg()(h, src_p, dst_p, ee)
        mlp = pl.pallas_call(
            functools.partial(_mlp_body, last=(l == L - 1)),
            out_shape=jax.ShapeDtypeStruct((N, F), _f32))
        h = mlp(h, parts[0, :N], parts[1, :N],
                (1.0 + gin_eps[l]).reshape(1, 1),
                mlp1_w[l], mlp1_b[l].reshape(1, -1),
                bn1_g[l].reshape(1, -1), bn1_b[l].reshape(1, -1),
                mlp2_w[l], mlp2_b[l].reshape(1, -1),
                bn_g[l].reshape(1, -1), bn_b[l].reshape(1, -1))
    return h
